# baseline (speedup 1.0000x reference)
import functools

import jax
import jax.numpy as jnp
from jax import lax
from jax.experimental import pallas as pl
from jax.experimental.pallas import tpu as pltpu
from jax.experimental.pallas import tpu_sc as plsc

_NC, _NS = 2, 16
_NW = _NC * _NS
_NBUF = 4
_LAG = 3


def kernel(x, embedding):
    B, S = x.shape
    D = embedding.shape[1]
    rows_w = S // _NW
    CH = 16
    n_ch = rows_w // CH

    mesh = plsc.VectorSubcoreMesh(core_axis_name="c", subcore_axis_name="s")

    @functools.partial(
        pl.kernel,
        out_type=jax.ShapeDtypeStruct((B, S, D), jnp.float32),
        mesh=mesh,
        scratch_types=(
            [pltpu.VMEM((CH, D), jnp.float32)] * _NBUF
            + [pltpu.SemaphoreType.DMA] * (2 * _NBUF)
        ),
    )
    def sc_copy(emb_hbm, out_hbm, *scratch):
        bufs = scratch[:_NBUF]
        rsems = scratch[_NBUF:2 * _NBUF]
        wsems = scratch[2 * _NBUF:]
        wid = lax.axis_index("s") * _NC + lax.axis_index("c")
        base = wid * rows_w

        def read(i):
            r = base + i * CH
            return pltpu.make_async_copy(
                emb_hbm.at[pl.ds(r, CH)], bufs[i % _NBUF], rsems[i % _NBUF])

        def writes(i):
            r = base + i * CH
            return [
                pltpu.make_async_copy(
                    bufs[i % _NBUF], out_hbm.at[b, pl.ds(r, CH)],
                    wsems[i % _NBUF])
                for b in range(B)
            ]

        for k in range(min(_NBUF, n_ch)):
            read(k).start()
        drained = 0
        for i in range(n_ch):
            read(i).wait()
            for w in writes(i):
                w.start()
            j = i - _LAG
            if j >= 0 and j + _NBUF < n_ch:
                for w in writes(j):
                    w.wait()
                drained = j + 1
                read(j + _NBUF).start()
        for j in range(drained, n_ch):
            for w in writes(j):
                w.wait()

    return sc_copy(embedding[:S])


# speedup vs baseline: 1.0585x; 1.0585x over previous
import functools

import jax
import jax.numpy as jnp
from jax import lax
from jax.experimental import pallas as pl
from jax.experimental.pallas import tpu as pltpu
from jax.experimental.pallas import tpu_sc as plsc

_NC, _NS = 2, 16
_NW = _NC * _NS
_NBUF = 3
_LAG = 2


def kernel(x, embedding):
    B, S = x.shape
    D = embedding.shape[1]
    rows_w = S // _NW
    CH = 32
    n_ch = rows_w // CH

    mesh = plsc.VectorSubcoreMesh(core_axis_name="c", subcore_axis_name="s")

    @functools.partial(
        pl.kernel,
        out_type=jax.ShapeDtypeStruct((B, S, D), jnp.float32),
        mesh=mesh,
        scratch_types=(
            [pltpu.VMEM((CH, D), jnp.float32)] * _NBUF
            + [pltpu.SemaphoreType.DMA] * (2 * _NBUF)
        ),
    )
    def sc_copy(emb_hbm, out_hbm, *scratch):
        bufs = scratch[:_NBUF]
        rsems = scratch[_NBUF:2 * _NBUF]
        wsems = scratch[2 * _NBUF:]
        wid = lax.axis_index("s") * _NC + lax.axis_index("c")
        base = wid * rows_w

        def read(i):
            r = base + i * CH
            return pltpu.make_async_copy(
                emb_hbm.at[pl.ds(r, CH)], bufs[i % _NBUF], rsems[i % _NBUF])

        def writes(i):
            r = base + i * CH
            return [
                pltpu.make_async_copy(
                    bufs[i % _NBUF], out_hbm.at[b, pl.ds(r, CH)],
                    wsems[i % _NBUF])
                for b in range(B)
            ]

        for k in range(min(_NBUF, n_ch)):
            read(k).start()
        drained = 0
        for i in range(n_ch):
            read(i).wait()
            for w in writes(i):
                w.start()
            j = i - _LAG
            if j >= 0 and j + _NBUF < n_ch:
                for w in writes(j):
                    w.wait()
                drained = j + 1
                read(j + _NBUF).start()
        for j in range(drained, n_ch):
            for w in writes(j):
                w.wait()

    return sc_copy(embedding[:S])
